# baseline (speedup 1.0000x reference)
import functools

import jax
import jax.numpy as jnp
from jax import lax
from jax.experimental import pallas as pl
from jax.experimental.pallas import tpu as pltpu
from jax.experimental.pallas import tpu_sc as plsc

N = 10000
E = 160000
C = 256
H = 512
CH = 128

NC, NS, L = 2, 16, 16
CHUNK = 128
SUB_CHUNKS = 80
IDX = 16
N_STAGES = SUB_CHUNKS // IDX
E_SUB = SUB_CHUNKS * CHUNK
E_PAD = E_SUB * NS
ROWS_SUB = 640
ACC_ROWS = ROWS_SUB * NS

R = 1000
BN_EPS = 1e-5


def _sc_segment_sum(xrows, srcp, dstp):
    mesh = plsc.VectorSubcoreMesh(core_axis_name="c", subcore_axis_name="s")

    @functools.partial(
        pl.kernel,
        out_type=jax.ShapeDtypeStruct((NC * ACC_ROWS, CH), jnp.float32),
        mesh=mesh,
        scratch_types=[
            pltpu.VMEM((2, IDX, CHUNK), jnp.int32),
            pltpu.VMEM((2, IDX, CHUNK), jnp.int32),
            pltpu.VMEM((CHUNK, CH), jnp.float32),
            pltpu.VMEM((CHUNK, CH), jnp.float32),
            pltpu.VMEM_SHARED((ACC_ROWS, CH), jnp.float32),
            pltpu.SemaphoreType.DMA,
            pltpu.SemaphoreType.DMA,
            pltpu.SemaphoreType.DMA,
            pltpu.SemaphoreType.DMA,
            pltpu.SemaphoreType.DMA,
        ],
    )
    def k(x_hbm, src_hbm, dst_hbm, out_hbm,
          src_v, dst_v, rows_a, rows_b, acc, sem_a, sem_b, sem_i,
          sem_sa, sem_sb):
        c = lax.axis_index("c")
        s = lax.axis_index("s")
        sbase = c * (E_PAD // CHUNK) + s * SUB_CHUNKS
        dbase = s * SUB_CHUNKS

        pltpu.sync_copy(src_hbm.at[pl.ds(sbase, IDX)], src_v.at[0])
        pltpu.sync_copy(dst_hbm.at[pl.ds(dbase, IDX)], dst_v.at[0])

        zero = jnp.zeros((L,), jnp.float32)

        @pl.loop(0, CHUNK)
        def _(r):
            @pl.loop(0, CH // L)
            def _(l):
                rows_a[r, pl.ds(l * L, L)] = zero

        @pl.loop(0, ROWS_SUB // CHUNK)
        def _(b):
            pltpu.sync_copy(
                rows_a, acc.at[pl.ds(s * ROWS_SUB + b * CHUNK, CHUNK)])

        plsc.subcore_barrier()

        for t in range(N_STAGES):
            slot = t % 2
            sv = src_v.at[slot]
            dv = dst_v.at[slot]
            if t > 0:
                pltpu.make_async_copy(
                    src_hbm.at[pl.ds(sbase + t * IDX, IDX)],
                    src_v.at[slot], sem_i).wait()
                pltpu.make_async_copy(
                    dst_hbm.at[pl.ds(dbase + t * IDX, IDX)],
                    dst_v.at[slot], sem_i).wait()
            if t + 1 < N_STAGES:
                pltpu.async_copy(
                    src_hbm.at[pl.ds(sbase + (t + 1) * IDX, IDX)],
                    src_v.at[1 - slot], sem_i)
                pltpu.async_copy(
                    dst_hbm.at[pl.ds(dbase + (t + 1) * IDX, IDX)],
                    dst_v.at[1 - slot], sem_i)

            pltpu.async_copy(x_hbm.at[sv.at[0]], rows_a, sem_a)

            @pl.loop(0, IDX, step=2)
            def _(j):
                pltpu.async_copy(x_hbm.at[sv.at[j + 1]], rows_b, sem_b)
                pltpu.make_async_copy(
                    x_hbm.at[sv.at[j]], rows_a, sem_a).wait()
                pltpu.async_copy(rows_a, acc.at[dv.at[j]], sem_sa, add=True)
                pltpu.make_async_copy(
                    x_hbm.at[sv.at[j + 1]], rows_b, sem_b).wait()
                pltpu.async_copy(rows_b, acc.at[dv.at[j + 1]], sem_sb,
                                 add=True)
                pltpu.make_async_copy(
                    rows_a, acc.at[dv.at[j]], sem_sa).wait()

                @pl.when(j + 2 < IDX)
                def _():
                    pltpu.async_copy(x_hbm.at[sv.at[j + 2]], rows_a, sem_a)

                pltpu.make_async_copy(
                    rows_b, acc.at[dv.at[j + 1]], sem_sb).wait()

        plsc.subcore_barrier()

        pltpu.sync_copy(
            acc.at[pl.ds(s * ROWS_SUB, ROWS_SUB)],
            out_hbm.at[pl.ds(c * ACC_ROWS + s * ROWS_SUB, ROWS_SUB)])

    return k(xrows, srcp, dstp)


def _mlp(scale, x, agg3, W1, b1, g1, be1, W2, b2, g2, be2):
    def body(sc_ref, x_ref, a_ref, w1_ref, b1_ref, g1_ref, be1_ref,
             w2_ref, b2_ref, g2_ref, be2_ref, o_ref,
             h_s, y2_s, G_s, hs_s, m1_s, i1_s, s2_s, q2_s, m2_s, i2_s):
        p = pl.program_id(0)
        i = pl.program_id(1)

        @pl.when(p == 0)
        def _():
            sc = sc_ref[0, 0]
            h = jnp.concatenate(
                [sc * x_ref[:, :CH] + a_ref[0],
                 sc * x_ref[:, CH:] + a_ref[1]], axis=1)
            h_s[pl.ds(i * R, R), :] = h
            G = lax.dot_general(h, h, (((0,), (0,)), ((), ())),
                                preferred_element_type=jnp.float32)
            cs = jnp.sum(h, axis=0, keepdims=True)

            @pl.when(i == 0)
            def _():
                G_s[...] = G
                hs_s[...] = cs

            @pl.when(i != 0)
            def _():
                G_s[...] += G
                hs_s[...] += cs

        @pl.when(p == 1)
        def _():
            @pl.when(i == 0)
            def _():
                w1 = w1_ref[...]
                b1v = b1_ref[...]
                sw = lax.dot_general(hs_s[...], w1, (((1,), (0,)), ((), ())),
                                     preferred_element_type=jnp.float32)
                gw = lax.dot_general(G_s[...], w1, (((1,), (0,)), ((), ())),
                                     preferred_element_type=jnp.float32)
                q1 = (jnp.sum(w1 * gw, axis=0, keepdims=True)
                      + 2.0 * b1v * sw + N * b1v * b1v)
                s1 = sw + N * b1v
                m = s1 * (1.0 / N)
                v = q1 * (1.0 / N) - m * m
                m1_s[...] = m
                i1_s[...] = lax.rsqrt(v + BN_EPS) * g1_ref[...]

            h = h_s[pl.ds(i * R, R), :]
            y1 = lax.dot_general(h, w1_ref[...], (((1,), (0,)), ((), ())),
                                 preferred_element_type=jnp.float32)
            y1 = y1 + b1_ref[...]
            h1 = jnp.maximum((y1 - m1_s[...]) * i1_s[...] + be1_ref[...], 0.0)
            y2 = lax.dot_general(h1, w2_ref[...], (((1,), (0,)), ((), ())),
                                 preferred_element_type=jnp.float32)
            y2 = y2 + b2_ref[...]
            y2_s[pl.ds(i * R, R), :] = y2
            cs = jnp.sum(y2, axis=0, keepdims=True)
            cq = jnp.sum(y2 * y2, axis=0, keepdims=True)

            @pl.when(i == 0)
            def _():
                s2_s[...] = cs
                q2_s[...] = cq

            @pl.when(i != 0)
            def _():
                s2_s[...] += cs
                q2_s[...] += cq

        @pl.when(p == 2)
        def _():
            @pl.when(i == 0)
            def _():
                m = s2_s[...] * (1.0 / N)
                v = q2_s[...] * (1.0 / N) - m * m
                m2_s[...] = m
                i2_s[...] = lax.rsqrt(v + BN_EPS) * g2_ref[...]

            y2 = y2_s[pl.ds(i * R, R), :]
            o_ref[...] = jnp.maximum(
                (y2 - m2_s[...]) * i2_s[...] + be2_ref[...], 0.0)

    zero2 = lambda p, i: (0, 0)
    return pl.pallas_call(
        body,
        grid=(3, N // R),
        in_specs=[
            pl.BlockSpec((1, 1), zero2),
            pl.BlockSpec((R, C), lambda p, i: (jnp.where(p == 0, i, 0), 0)),
            pl.BlockSpec((2, R, CH),
                         lambda p, i: (0, jnp.where(p == 0, i, 0), 0)),
            pl.BlockSpec((C, H), zero2),
            pl.BlockSpec((1, H), zero2),
            pl.BlockSpec((1, H), zero2),
            pl.BlockSpec((1, H), zero2),
            pl.BlockSpec((H, C), zero2),
            pl.BlockSpec((1, C), zero2),
            pl.BlockSpec((1, C), zero2),
            pl.BlockSpec((1, C), zero2),
        ],
        out_specs=pl.BlockSpec((R, C),
                               lambda p, i: (jnp.where(p == 2, i, 0), 0)),
        out_shape=jax.ShapeDtypeStruct((N, C), jnp.float32),
        scratch_shapes=[
            pltpu.VMEM((N, C), jnp.float32),
            pltpu.VMEM((N, C), jnp.float32),
            pltpu.VMEM((C, C), jnp.float32),
            pltpu.VMEM((1, C), jnp.float32),
            pltpu.VMEM((1, H), jnp.float32),
            pltpu.VMEM((1, H), jnp.float32),
            pltpu.VMEM((1, C), jnp.float32),
            pltpu.VMEM((1, C), jnp.float32),
            pltpu.VMEM((1, C), jnp.float32),
            pltpu.VMEM((1, C), jnp.float32),
        ],
    )(scale, x, agg3, W1, b1, g1, be1, W2, b2, g2, be2)


def kernel(x, edge_index, eps, W1, b1, g1, be1, W2, b2, g2, be2):
    src = edge_index[0]
    dst = edge_index[1]

    pad_n = E_PAD - E
    spread = jnp.arange(pad_n, dtype=jnp.int32) % 8
    src_p = jnp.concatenate([src, spread]).reshape(E_PAD // CHUNK, CHUNK)
    dst_p = jnp.concatenate(
        [dst, N + spread]).reshape(E_PAD // CHUNK, CHUNK)
    srcp = jnp.concatenate([2 * src_p, 2 * src_p + 1], axis=0)

    agg = _sc_segment_sum(x.reshape(2 * N, CH), srcp, dst_p)
    agg3 = agg.reshape(NC, ACC_ROWS, CH)

    scale = (1.0 + eps).reshape(1, 1).astype(jnp.float32)
    return _mlp(scale, x, agg3, W1, b1.reshape(1, H), g1.reshape(1, H),
                be1.reshape(1, H), W2, b2.reshape(1, C), g2.reshape(1, C),
                be2.reshape(1, C))


# speedup vs baseline: 1.1320x; 1.1320x over previous
import functools

import jax
import jax.numpy as jnp
from jax import lax
from jax.experimental import pallas as pl
from jax.experimental.pallas import tpu as pltpu
from jax.experimental.pallas import tpu_sc as plsc

N = 10000
E = 160000
C = 256
H = 512
CH = 128

NC, NS, L = 2, 16, 16
CHUNK = 128
SUB_CHUNKS = 80
IDX = 16
N_STAGES = SUB_CHUNKS // IDX
E_SUB = SUB_CHUNKS * CHUNK
E_PAD = E_SUB * NS
ROWS_SUB = 640
ACC_ROWS = ROWS_SUB * NS

R = 1000
BN_EPS = 1e-5


def _sc_segment_sum(xrows, srcp, dstp):
    mesh = plsc.VectorSubcoreMesh(core_axis_name="c", subcore_axis_name="s")

    @functools.partial(
        pl.kernel,
        out_type=jax.ShapeDtypeStruct((NC * ACC_ROWS, CH), jnp.float32),
        mesh=mesh,
        scratch_types=[
            pltpu.VMEM((2, IDX, CHUNK), jnp.int32),
            pltpu.VMEM((2, IDX, CHUNK), jnp.int32),
            pltpu.VMEM((CHUNK, CH), jnp.float32),
            pltpu.VMEM((CHUNK, CH), jnp.float32),
            pltpu.VMEM_SHARED((ACC_ROWS, CH), jnp.float32),
            pltpu.SemaphoreType.DMA,
            pltpu.SemaphoreType.DMA,
            pltpu.SemaphoreType.DMA,
            pltpu.SemaphoreType.DMA,
            pltpu.SemaphoreType.DMA,
        ],
    )
    def k(x_hbm, src_hbm, dst_hbm, out_hbm,
          src_v, dst_v, rows_a, rows_b, acc, sem_a, sem_b, sem_i,
          sem_sa, sem_sb):
        c = lax.axis_index("c")
        s = lax.axis_index("s")
        sbase = c * (E_PAD // CHUNK) + s * SUB_CHUNKS
        dbase = s * SUB_CHUNKS

        pltpu.sync_copy(src_hbm.at[pl.ds(sbase, IDX)], src_v.at[0])
        pltpu.sync_copy(dst_hbm.at[pl.ds(dbase, IDX)], dst_v.at[0])

        zero = jnp.zeros((L,), jnp.float32)

        @pl.loop(0, CHUNK)
        def _(r):
            @pl.loop(0, CH // L)
            def _(l):
                rows_a[r, pl.ds(l * L, L)] = zero

        @pl.loop(0, ROWS_SUB // CHUNK)
        def _(b):
            pltpu.sync_copy(
                rows_a, acc.at[pl.ds(s * ROWS_SUB + b * CHUNK, CHUNK)])

        plsc.subcore_barrier()

        for t in range(N_STAGES):
            slot = t % 2
            sv = src_v.at[slot]
            dv = dst_v.at[slot]
            if t > 0:
                pltpu.make_async_copy(
                    src_hbm.at[pl.ds(sbase + t * IDX, IDX)],
                    src_v.at[slot], sem_i).wait()
                pltpu.make_async_copy(
                    dst_hbm.at[pl.ds(dbase + t * IDX, IDX)],
                    dst_v.at[slot], sem_i).wait()
            if t + 1 < N_STAGES:
                pltpu.async_copy(
                    src_hbm.at[pl.ds(sbase + (t + 1) * IDX, IDX)],
                    src_v.at[1 - slot], sem_i)
                pltpu.async_copy(
                    dst_hbm.at[pl.ds(dbase + (t + 1) * IDX, IDX)],
                    dst_v.at[1 - slot], sem_i)

            pltpu.async_copy(x_hbm.at[sv.at[0]], rows_a, sem_a)

            @pl.loop(0, IDX, step=2)
            def _(j):
                pltpu.async_copy(x_hbm.at[sv.at[j + 1]], rows_b, sem_b)
                pltpu.make_async_copy(
                    x_hbm.at[sv.at[j]], rows_a, sem_a).wait()
                pltpu.sync_copy(rows_a, acc.at[dv.at[j]], add=True)

                @pl.when(j + 2 < IDX)
                def _():
                    pltpu.async_copy(x_hbm.at[sv.at[j + 2]], rows_a, sem_a)

                pltpu.make_async_copy(
                    x_hbm.at[sv.at[j + 1]], rows_b, sem_b).wait()
                pltpu.sync_copy(rows_b, acc.at[dv.at[j + 1]], add=True)

        plsc.subcore_barrier()

        pltpu.sync_copy(
            acc.at[pl.ds(s * ROWS_SUB, ROWS_SUB)],
            out_hbm.at[pl.ds(c * ACC_ROWS + s * ROWS_SUB, ROWS_SUB)])

    return k(xrows, srcp, dstp)


def _mlp(scale, x, agg3, W1, b1, g1, be1, W2, b2, g2, be2):
    def body(sc_ref, x_ref, a_ref, w1_ref, b1_ref, g1_ref, be1_ref,
             w2_ref, b2_ref, g2_ref, be2_ref, o_ref,
             h_s, y2_s, G_s, hs_s, m1_s, i1_s, s2_s, q2_s, m2_s, i2_s):
        p = pl.program_id(0)
        i = pl.program_id(1)

        @pl.when(p == 0)
        def _():
            sc = sc_ref[0, 0]
            h = jnp.concatenate(
                [sc * x_ref[:, :CH] + a_ref[0],
                 sc * x_ref[:, CH:] + a_ref[1]], axis=1)
            h_s[pl.ds(i * R, R), :] = h
            G = lax.dot_general(h, h, (((0,), (0,)), ((), ())),
                                preferred_element_type=jnp.float32)
            cs = jnp.sum(h, axis=0, keepdims=True)

            @pl.when(i == 0)
            def _():
                G_s[...] = G
                hs_s[...] = cs

            @pl.when(i != 0)
            def _():
                G_s[...] += G
                hs_s[...] += cs

        @pl.when(p == 1)
        def _():
            @pl.when(i == 0)
            def _():
                w1 = w1_ref[...]
                b1v = b1_ref[...]
                sw = lax.dot_general(hs_s[...], w1, (((1,), (0,)), ((), ())),
                                     preferred_element_type=jnp.float32)
                gw = lax.dot_general(G_s[...], w1, (((1,), (0,)), ((), ())),
                                     preferred_element_type=jnp.float32)
                q1 = (jnp.sum(w1 * gw, axis=0, keepdims=True)
                      + 2.0 * b1v * sw + N * b1v * b1v)
                s1 = sw + N * b1v
                m = s1 * (1.0 / N)
                v = q1 * (1.0 / N) - m * m
                m1_s[...] = m
                i1_s[...] = lax.rsqrt(v + BN_EPS) * g1_ref[...]

            h = h_s[pl.ds(i * R, R), :]
            y1 = lax.dot_general(h, w1_ref[...], (((1,), (0,)), ((), ())),
                                 preferred_element_type=jnp.float32)
            y1 = y1 + b1_ref[...]
            h1 = jnp.maximum((y1 - m1_s[...]) * i1_s[...] + be1_ref[...], 0.0)
            y2 = lax.dot_general(h1, w2_ref[...], (((1,), (0,)), ((), ())),
                                 preferred_element_type=jnp.float32)
            y2 = y2 + b2_ref[...]
            y2_s[pl.ds(i * R, R), :] = y2
            cs = jnp.sum(y2, axis=0, keepdims=True)
            cq = jnp.sum(y2 * y2, axis=0, keepdims=True)

            @pl.when(i == 0)
            def _():
                s2_s[...] = cs
                q2_s[...] = cq

            @pl.when(i != 0)
            def _():
                s2_s[...] += cs
                q2_s[...] += cq

        @pl.when(p == 2)
        def _():
            @pl.when(i == 0)
            def _():
                m = s2_s[...] * (1.0 / N)
                v = q2_s[...] * (1.0 / N) - m * m
                m2_s[...] = m
                i2_s[...] = lax.rsqrt(v + BN_EPS) * g2_ref[...]

            y2 = y2_s[pl.ds(i * R, R), :]
            o_ref[...] = jnp.maximum(
                (y2 - m2_s[...]) * i2_s[...] + be2_ref[...], 0.0)

    zero2 = lambda p, i: (0, 0)
    return pl.pallas_call(
        body,
        grid=(3, N // R),
        in_specs=[
            pl.BlockSpec((1, 1), zero2),
            pl.BlockSpec((R, C), lambda p, i: (jnp.where(p == 0, i, 0), 0)),
            pl.BlockSpec((2, R, CH),
                         lambda p, i: (0, jnp.where(p == 0, i, 0), 0)),
            pl.BlockSpec((C, H), zero2),
            pl.BlockSpec((1, H), zero2),
            pl.BlockSpec((1, H), zero2),
            pl.BlockSpec((1, H), zero2),
            pl.BlockSpec((H, C), zero2),
            pl.BlockSpec((1, C), zero2),
            pl.BlockSpec((1, C), zero2),
            pl.BlockSpec((1, C), zero2),
        ],
        out_specs=pl.BlockSpec((R, C),
                               lambda p, i: (jnp.where(p == 2, i, 0), 0)),
        out_shape=jax.ShapeDtypeStruct((N, C), jnp.float32),
        scratch_shapes=[
            pltpu.VMEM((N, C), jnp.float32),
            pltpu.VMEM((N, C), jnp.float32),
            pltpu.VMEM((C, C), jnp.float32),
            pltpu.VMEM((1, C), jnp.float32),
            pltpu.VMEM((1, H), jnp.float32),
            pltpu.VMEM((1, H), jnp.float32),
            pltpu.VMEM((1, C), jnp.float32),
            pltpu.VMEM((1, C), jnp.float32),
            pltpu.VMEM((1, C), jnp.float32),
            pltpu.VMEM((1, C), jnp.float32),
        ],
    )(scale, x, agg3, W1, b1, g1, be1, W2, b2, g2, be2)


def kernel(x, edge_index, eps, W1, b1, g1, be1, W2, b2, g2, be2):
    src = edge_index[0]
    dst = edge_index[1]

    pad_n = E_PAD - E
    spread = jnp.arange(pad_n, dtype=jnp.int32) % 8
    src_p = jnp.concatenate([src, spread]).reshape(E_PAD // CHUNK, CHUNK)
    dst_p = jnp.concatenate(
        [dst, N + spread]).reshape(E_PAD // CHUNK, CHUNK)
    srcp = jnp.concatenate([2 * src_p, 2 * src_p + 1], axis=0)

    agg = _sc_segment_sum(x.reshape(2 * N, CH), srcp, dst_p)
    agg3 = agg.reshape(NC, ACC_ROWS, CH)

    scale = (1.0 + eps).reshape(1, 1).astype(jnp.float32)
    return _mlp(scale, x, agg3, W1, b1.reshape(1, H), g1.reshape(1, H),
                be1.reshape(1, H), W2, b2.reshape(1, C), g2.reshape(1, C),
                be2.reshape(1, C))


# speedup vs baseline: 1.1527x; 1.0183x over previous
import functools

import jax
import jax.numpy as jnp
from jax import lax
from jax.experimental import pallas as pl
from jax.experimental.pallas import tpu as pltpu
from jax.experimental.pallas import tpu_sc as plsc

N = 10000
E = 160000
C = 256
H = 512
CH = 128

NC, NS, L = 2, 16, 16
CHUNK = 128
SUB_CHUNKS = 80
IDX = 16
N_STAGES = SUB_CHUNKS // IDX
E_SUB = SUB_CHUNKS * CHUNK
E_PAD = E_SUB * NS
ROWS_SUB = 640
ACC_ROWS = ROWS_SUB * NS

R = 1000
BN_EPS = 1e-5


def _sc_segment_sum(xrows, srcp, dstp):
    mesh = plsc.VectorSubcoreMesh(core_axis_name="c", subcore_axis_name="s")

    @functools.partial(
        pl.kernel,
        out_type=jax.ShapeDtypeStruct((NC * ACC_ROWS, CH), jnp.float32),
        mesh=mesh,
        scratch_types=[
            pltpu.VMEM((2, IDX, CHUNK), jnp.int32),
            pltpu.VMEM((2, IDX, CHUNK), jnp.int32),
            pltpu.VMEM((CHUNK, CH), jnp.float32),
            pltpu.VMEM((CHUNK, CH), jnp.float32),
            pltpu.VMEM_SHARED((ACC_ROWS, CH), jnp.float32),
            pltpu.SemaphoreType.DMA,
            pltpu.SemaphoreType.DMA,
            pltpu.SemaphoreType.DMA,
            pltpu.SemaphoreType.DMA,
            pltpu.SemaphoreType.DMA,
        ],
    )
    def k(x_hbm, src_hbm, dst_hbm, out_hbm,
          src_v, dst_v, rows_a, rows_b, acc, sem_a, sem_b, sem_i,
          sem_sa, sem_sb):
        c = lax.axis_index("c")
        s = lax.axis_index("s")
        sbase = s * SUB_CHUNKS
        dbase = s * SUB_CHUNKS

        pltpu.sync_copy(src_hbm.at[pl.ds(sbase, IDX)], src_v.at[0])
        pltpu.sync_copy(dst_hbm.at[pl.ds(dbase, IDX)], dst_v.at[0])

        zero = jnp.zeros((L,), jnp.float32)

        @pl.loop(0, CHUNK)
        def _(r):
            @pl.loop(0, CH // L)
            def _(l):
                rows_a[r, pl.ds(l * L, L)] = zero

        @pl.loop(0, ROWS_SUB // CHUNK)
        def _(b):
            pltpu.sync_copy(
                rows_a, acc.at[pl.ds(s * ROWS_SUB + b * CHUNK, CHUNK)])

        plsc.subcore_barrier()
        xv = x_hbm.at[:, pl.ds(c * CH, CH)]

        for t in range(N_STAGES):
            slot = t % 2
            sv = src_v.at[slot]
            dv = dst_v.at[slot]
            if t > 0:
                pltpu.make_async_copy(
                    src_hbm.at[pl.ds(sbase + t * IDX, IDX)],
                    src_v.at[slot], sem_i).wait()
                pltpu.make_async_copy(
                    dst_hbm.at[pl.ds(dbase + t * IDX, IDX)],
                    dst_v.at[slot], sem_i).wait()
            if t + 1 < N_STAGES:
                pltpu.async_copy(
                    src_hbm.at[pl.ds(sbase + (t + 1) * IDX, IDX)],
                    src_v.at[1 - slot], sem_i)
                pltpu.async_copy(
                    dst_hbm.at[pl.ds(dbase + (t + 1) * IDX, IDX)],
                    dst_v.at[1 - slot], sem_i)

            pltpu.async_copy(xv.at[sv.at[0]], rows_a, sem_a)

            @pl.loop(0, IDX, step=2)
            def _(j):
                pltpu.async_copy(xv.at[sv.at[j + 1]], rows_b, sem_b)
                pltpu.make_async_copy(
                    xv.at[sv.at[j]], rows_a, sem_a).wait()
                pltpu.sync_copy(rows_a, acc.at[dv.at[j]], add=True)

                @pl.when(j + 2 < IDX)
                def _():
                    pltpu.async_copy(xv.at[sv.at[j + 2]], rows_a, sem_a)

                pltpu.make_async_copy(
                    xv.at[sv.at[j + 1]], rows_b, sem_b).wait()
                pltpu.sync_copy(rows_b, acc.at[dv.at[j + 1]], add=True)

        plsc.subcore_barrier()

        pltpu.sync_copy(
            acc.at[pl.ds(s * ROWS_SUB, ROWS_SUB)],
            out_hbm.at[pl.ds(c * ACC_ROWS + s * ROWS_SUB, ROWS_SUB)])

    return k(xrows, srcp, dstp)


def _mlp(scale, x, agg3, W1, b1, g1, be1, W2, b2, g2, be2):
    def body(sc_ref, x_ref, a_ref, w1_ref, b1_ref, g1_ref, be1_ref,
             w2_ref, b2_ref, g2_ref, be2_ref, o_ref,
             h_s, y2_s, G_s, hs_s, m1_s, i1_s, s2_s, q2_s, m2_s, i2_s):
        p = pl.program_id(0)
        i = pl.program_id(1)

        @pl.when(p == 0)
        def _():
            sc = sc_ref[0, 0]
            h = jnp.concatenate(
                [sc * x_ref[:, :CH] + a_ref[0],
                 sc * x_ref[:, CH:] + a_ref[1]], axis=1)
            h_s[pl.ds(i * R, R), :] = h
            G = lax.dot_general(h, h, (((0,), (0,)), ((), ())),
                                preferred_element_type=jnp.float32)
            cs = jnp.sum(h, axis=0, keepdims=True)

            @pl.when(i == 0)
            def _():
                G_s[...] = G
                hs_s[...] = cs

            @pl.when(i != 0)
            def _():
                G_s[...] += G
                hs_s[...] += cs

        @pl.when(p == 1)
        def _():
            @pl.when(i == 0)
            def _():
                w1 = w1_ref[...]
                b1v = b1_ref[...]
                sw = lax.dot_general(hs_s[...], w1, (((1,), (0,)), ((), ())),
                                     preferred_element_type=jnp.float32)
                gw = lax.dot_general(G_s[...], w1, (((1,), (0,)), ((), ())),
                                     preferred_element_type=jnp.float32)
                q1 = (jnp.sum(w1 * gw, axis=0, keepdims=True)
                      + 2.0 * b1v * sw + N * b1v * b1v)
                s1 = sw + N * b1v
                m = s1 * (1.0 / N)
                v = q1 * (1.0 / N) - m * m
                m1_s[...] = m
                i1_s[...] = lax.rsqrt(v + BN_EPS) * g1_ref[...]

            h = h_s[pl.ds(i * R, R), :]
            y1 = lax.dot_general(h, w1_ref[...], (((1,), (0,)), ((), ())),
                                 preferred_element_type=jnp.float32)
            y1 = y1 + b1_ref[...]
            h1 = jnp.maximum((y1 - m1_s[...]) * i1_s[...] + be1_ref[...], 0.0)
            y2 = lax.dot_general(h1, w2_ref[...], (((1,), (0,)), ((), ())),
                                 preferred_element_type=jnp.float32)
            y2 = y2 + b2_ref[...]
            y2_s[pl.ds(i * R, R), :] = y2
            cs = jnp.sum(y2, axis=0, keepdims=True)
            cq = jnp.sum(y2 * y2, axis=0, keepdims=True)

            @pl.when(i == 0)
            def _():
                s2_s[...] = cs
                q2_s[...] = cq

            @pl.when(i != 0)
            def _():
                s2_s[...] += cs
                q2_s[...] += cq

        @pl.when(p == 2)
        def _():
            @pl.when(i == 0)
            def _():
                m = s2_s[...] * (1.0 / N)
                v = q2_s[...] * (1.0 / N) - m * m
                m2_s[...] = m
                i2_s[...] = lax.rsqrt(v + BN_EPS) * g2_ref[...]

            y2 = y2_s[pl.ds(i * R, R), :]
            o_ref[...] = jnp.maximum(
                (y2 - m2_s[...]) * i2_s[...] + be2_ref[...], 0.0)

    zero2 = lambda p, i: (0, 0)
    return pl.pallas_call(
        body,
        grid=(3, N // R),
        in_specs=[
            pl.BlockSpec((1, 1), zero2),
            pl.BlockSpec((R, C), lambda p, i: (jnp.where(p == 0, i, 0), 0)),
            pl.BlockSpec((2, R, CH),
                         lambda p, i: (0, jnp.where(p == 0, i, 0), 0)),
            pl.BlockSpec((C, H), zero2),
            pl.BlockSpec((1, H), zero2),
            pl.BlockSpec((1, H), zero2),
            pl.BlockSpec((1, H), zero2),
            pl.BlockSpec((H, C), zero2),
            pl.BlockSpec((1, C), zero2),
            pl.BlockSpec((1, C), zero2),
            pl.BlockSpec((1, C), zero2),
        ],
        out_specs=pl.BlockSpec((R, C),
                               lambda p, i: (jnp.where(p == 2, i, 0), 0)),
        out_shape=jax.ShapeDtypeStruct((N, C), jnp.float32),
        scratch_shapes=[
            pltpu.VMEM((N, C), jnp.float32),
            pltpu.VMEM((N, C), jnp.float32),
            pltpu.VMEM((C, C), jnp.float32),
            pltpu.VMEM((1, C), jnp.float32),
            pltpu.VMEM((1, H), jnp.float32),
            pltpu.VMEM((1, H), jnp.float32),
            pltpu.VMEM((1, C), jnp.float32),
            pltpu.VMEM((1, C), jnp.float32),
            pltpu.VMEM((1, C), jnp.float32),
            pltpu.VMEM((1, C), jnp.float32),
        ],
    )(scale, x, agg3, W1, b1, g1, be1, W2, b2, g2, be2)


def kernel(x, edge_index, eps, W1, b1, g1, be1, W2, b2, g2, be2):
    src = edge_index[0]
    dst = edge_index[1]

    pad_n = E_PAD - E
    spread = jnp.arange(pad_n, dtype=jnp.int32) % 8
    src_p = jnp.concatenate([src, N + spread]).reshape(E_PAD // CHUNK, CHUNK)
    dst_p = jnp.concatenate(
        [dst, N + spread]).reshape(E_PAD // CHUNK, CHUNK)

    xpad = jnp.concatenate([x, jnp.zeros((8, C), jnp.float32)])
    agg = _sc_segment_sum(xpad, src_p, dst_p)
    agg3 = agg.reshape(NC, ACC_ROWS, CH)

    scale = (1.0 + eps).reshape(1, 1).astype(jnp.float32)
    return _mlp(scale, x, agg3, W1, b1.reshape(1, H), g1.reshape(1, H),
                be1.reshape(1, H), W2, b2.reshape(1, C), g2.reshape(1, C),
                be2.reshape(1, C))


# speedup vs baseline: 1.1638x; 1.0096x over previous
import functools

import jax
import jax.numpy as jnp
from jax import lax
from jax.experimental import pallas as pl
from jax.experimental.pallas import tpu as pltpu
from jax.experimental.pallas import tpu_sc as plsc

N = 10000
E = 160000
C = 256
H = 512
CH = 128

NC, NS, L = 2, 16, 16
CHUNK = 128
SUB_CHUNKS = 80
IDX = 16
N_STAGES = SUB_CHUNKS // IDX
E_SUB = SUB_CHUNKS * CHUNK
E_PAD = E_SUB * NS
ROWS_SUB = 640
ACC_ROWS = ROWS_SUB * NS

R = 1000
BN_EPS = 1e-5


def _sc_segment_sum(xrows, srcp, dstp):
    mesh = plsc.VectorSubcoreMesh(core_axis_name="c", subcore_axis_name="s")

    @functools.partial(
        pl.kernel,
        out_type=jax.ShapeDtypeStruct((NC * ACC_ROWS, CH), jnp.float32),
        mesh=mesh,
        scratch_types=[
            pltpu.VMEM((2, IDX, CHUNK), jnp.int32),
            pltpu.VMEM((2, IDX, CHUNK), jnp.int32),
            pltpu.VMEM((CHUNK, CH), jnp.float32),
            pltpu.VMEM((CHUNK, CH), jnp.float32),
            pltpu.VMEM_SHARED((ACC_ROWS, CH), jnp.float32),
            pltpu.SemaphoreType.DMA,
            pltpu.SemaphoreType.DMA,
            pltpu.SemaphoreType.DMA,
        ],
    )
    def k(x_hbm, src_hbm, dst_hbm, out_hbm,
          src_v, dst_v, rows_a, rows_b, acc, sem_a, sem_b, sem_i):
        c = lax.axis_index("c")
        s = lax.axis_index("s")
        sbase = s * SUB_CHUNKS
        dbase = s * SUB_CHUNKS

        pltpu.async_copy(src_hbm.at[pl.ds(sbase, IDX)], src_v.at[0], sem_i)
        pltpu.async_copy(dst_hbm.at[pl.ds(dbase, IDX)], dst_v.at[0], sem_i)

        zero = jnp.zeros((L,), jnp.float32)

        @pl.loop(0, CHUNK)
        def _(r):
            @pl.loop(0, CH // L)
            def _(l):
                rows_a[r, pl.ds(l * L, L)] = zero

        for b in range(ROWS_SUB // CHUNK):
            pltpu.async_copy(
                rows_a, acc.at[pl.ds(s * ROWS_SUB + b * CHUNK, CHUNK)], sem_b)
        for b in range(ROWS_SUB // CHUNK):
            pltpu.make_async_copy(
                rows_a, acc.at[pl.ds(s * ROWS_SUB + b * CHUNK, CHUNK)],
                sem_b).wait()
        pltpu.make_async_copy(
            src_hbm.at[pl.ds(sbase, IDX)], src_v.at[0], sem_i).wait()
        pltpu.make_async_copy(
            dst_hbm.at[pl.ds(dbase, IDX)], dst_v.at[0], sem_i).wait()

        plsc.subcore_barrier()
        xv = x_hbm.at[:, pl.ds(c * CH, CH)]

        for t in range(N_STAGES):
            slot = t % 2
            sv = src_v.at[slot]
            dv = dst_v.at[slot]
            if t > 0:
                pltpu.make_async_copy(
                    src_hbm.at[pl.ds(sbase + t * IDX, IDX)],
                    src_v.at[slot], sem_i).wait()
                pltpu.make_async_copy(
                    dst_hbm.at[pl.ds(dbase + t * IDX, IDX)],
                    dst_v.at[slot], sem_i).wait()
            if t + 1 < N_STAGES:
                pltpu.async_copy(
                    src_hbm.at[pl.ds(sbase + (t + 1) * IDX, IDX)],
                    src_v.at[1 - slot], sem_i)
                pltpu.async_copy(
                    dst_hbm.at[pl.ds(dbase + (t + 1) * IDX, IDX)],
                    dst_v.at[1 - slot], sem_i)

            pltpu.async_copy(xv.at[sv.at[0]], rows_a, sem_a)

            @pl.loop(0, IDX, step=2)
            def _(j):
                pltpu.async_copy(xv.at[sv.at[j + 1]], rows_b, sem_b)
                pltpu.make_async_copy(
                    xv.at[sv.at[j]], rows_a, sem_a).wait()
                pltpu.sync_copy(rows_a, acc.at[dv.at[j]], add=True)

                @pl.when(j + 2 < IDX)
                def _():
                    pltpu.async_copy(xv.at[sv.at[j + 2]], rows_a, sem_a)

                pltpu.make_async_copy(
                    xv.at[sv.at[j + 1]], rows_b, sem_b).wait()
                pltpu.sync_copy(rows_b, acc.at[dv.at[j + 1]], add=True)

        plsc.subcore_barrier()

        pltpu.sync_copy(
            acc.at[pl.ds(s * ROWS_SUB, ROWS_SUB)],
            out_hbm.at[pl.ds(c * ACC_ROWS + s * ROWS_SUB, ROWS_SUB)])

    return k(xrows, srcp, dstp)


def _mlp(scale, x, agg3, W1, b1, g1, be1, W2, b2, g2, be2):
    def body(sc_ref, x_ref, a_ref, w1_ref, b1_ref, g1_ref, be1_ref,
             w2_ref, b2_ref, g2_ref, be2_ref, o_ref,
             h_s, y2_s, G_s, hs_s, m1_s, i1_s, s2_s, q2_s, m2_s, i2_s):
        p = pl.program_id(0)
        i = pl.program_id(1)

        @pl.when(p == 0)
        def _():
            sc = sc_ref[0, 0]
            h = jnp.concatenate(
                [sc * x_ref[:, :CH] + a_ref[0],
                 sc * x_ref[:, CH:] + a_ref[1]], axis=1)
            h_s[pl.ds(i * R, R), :] = h
            G = lax.dot_general(h, h, (((0,), (0,)), ((), ())),
                                preferred_element_type=jnp.float32)
            cs = jnp.sum(h, axis=0, keepdims=True)

            @pl.when(i == 0)
            def _():
                G_s[...] = G
                hs_s[...] = cs

            @pl.when(i != 0)
            def _():
                G_s[...] += G
                hs_s[...] += cs

        @pl.when(p == 1)
        def _():
            @pl.when(i == 0)
            def _():
                w1 = w1_ref[...]
                b1v = b1_ref[...]
                sw = lax.dot_general(hs_s[...], w1, (((1,), (0,)), ((), ())),
                                     preferred_element_type=jnp.float32)
                gw = lax.dot_general(G_s[...], w1, (((1,), (0,)), ((), ())),
                                     preferred_element_type=jnp.float32)
                q1 = (jnp.sum(w1 * gw, axis=0, keepdims=True)
                      + 2.0 * b1v * sw + N * b1v * b1v)
                s1 = sw + N * b1v
                m = s1 * (1.0 / N)
                v = q1 * (1.0 / N) - m * m
                m1_s[...] = m
                i1_s[...] = lax.rsqrt(v + BN_EPS) * g1_ref[...]

            h = h_s[pl.ds(i * R, R), :]
            y1 = lax.dot_general(h, w1_ref[...], (((1,), (0,)), ((), ())),
                                 preferred_element_type=jnp.float32)
            y1 = y1 + b1_ref[...]
            h1 = jnp.maximum((y1 - m1_s[...]) * i1_s[...] + be1_ref[...], 0.0)
            y2 = lax.dot_general(h1, w2_ref[...], (((1,), (0,)), ((), ())),
                                 preferred_element_type=jnp.float32)
            y2 = y2 + b2_ref[...]
            y2_s[pl.ds(i * R, R), :] = y2
            cs = jnp.sum(y2, axis=0, keepdims=True)
            cq = jnp.sum(y2 * y2, axis=0, keepdims=True)

            @pl.when(i == 0)
            def _():
                s2_s[...] = cs
                q2_s[...] = cq

            @pl.when(i != 0)
            def _():
                s2_s[...] += cs
                q2_s[...] += cq

        @pl.when(p == 2)
        def _():
            @pl.when(i == 0)
            def _():
                m = s2_s[...] * (1.0 / N)
                v = q2_s[...] * (1.0 / N) - m * m
                m2_s[...] = m
                i2_s[...] = lax.rsqrt(v + BN_EPS) * g2_ref[...]

            y2 = y2_s[pl.ds(i * R, R), :]
            o_ref[...] = jnp.maximum(
                (y2 - m2_s[...]) * i2_s[...] + be2_ref[...], 0.0)

    zero2 = lambda p, i: (0, 0)
    return pl.pallas_call(
        body,
        grid=(3, N // R),
        in_specs=[
            pl.BlockSpec((1, 1), zero2),
            pl.BlockSpec((R, C), lambda p, i: (jnp.where(p == 0, i, 0), 0)),
            pl.BlockSpec((2, R, CH),
                         lambda p, i: (0, jnp.where(p == 0, i, 0), 0)),
            pl.BlockSpec((C, H), zero2),
            pl.BlockSpec((1, H), zero2),
            pl.BlockSpec((1, H), zero2),
            pl.BlockSpec((1, H), zero2),
            pl.BlockSpec((H, C), zero2),
            pl.BlockSpec((1, C), zero2),
            pl.BlockSpec((1, C), zero2),
            pl.BlockSpec((1, C), zero2),
        ],
        out_specs=pl.BlockSpec((R, C),
                               lambda p, i: (jnp.where(p == 2, i, 0), 0)),
        out_shape=jax.ShapeDtypeStruct((N, C), jnp.float32),
        scratch_shapes=[
            pltpu.VMEM((N, C), jnp.float32),
            pltpu.VMEM((N, C), jnp.float32),
            pltpu.VMEM((C, C), jnp.float32),
            pltpu.VMEM((1, C), jnp.float32),
            pltpu.VMEM((1, H), jnp.float32),
            pltpu.VMEM((1, H), jnp.float32),
            pltpu.VMEM((1, C), jnp.float32),
            pltpu.VMEM((1, C), jnp.float32),
            pltpu.VMEM((1, C), jnp.float32),
            pltpu.VMEM((1, C), jnp.float32),
        ],
    )(scale, x, agg3, W1, b1, g1, be1, W2, b2, g2, be2)


def kernel(x, edge_index, eps, W1, b1, g1, be1, W2, b2, g2, be2):
    src = edge_index[0]
    dst = edge_index[1]

    pad_n = E_PAD - E
    spread = jnp.arange(pad_n, dtype=jnp.int32) % 8
    src_p = jnp.concatenate([src, N + spread]).reshape(E_PAD // CHUNK, CHUNK)
    dst_p = jnp.concatenate(
        [dst, N + spread]).reshape(E_PAD // CHUNK, CHUNK)

    xpad = jnp.concatenate([x, jnp.zeros((8, C), jnp.float32)])
    agg = _sc_segment_sum(xpad, src_p, dst_p)
    agg3 = agg.reshape(NC, ACC_ROWS, CH)

    scale = (1.0 + eps).reshape(1, 1).astype(jnp.float32)
    return _mlp(scale, x, agg3, W1, b1.reshape(1, H), g1.reshape(1, H),
                be1.reshape(1, H), W2, b2.reshape(1, C), g2.reshape(1, C),
                be2.reshape(1, C))


# speedup vs baseline: 1.2554x; 1.0787x over previous
import functools

import jax
import jax.numpy as jnp
from jax import lax
from jax.experimental import pallas as pl
from jax.experimental.pallas import tpu as pltpu
from jax.experimental.pallas import tpu_sc as plsc

N = 10000
E = 160000
C = 256
H = 512
CH = 128

NC, NS, L = 2, 16, 16
CHUNK = 128
SUB_CHUNKS = 80
IDX = 16
N_STAGES = SUB_CHUNKS // IDX
E_SUB = SUB_CHUNKS * CHUNK
E_PAD = E_SUB * NS
ROWS_SUB = 640
ACC_ROWS = ROWS_SUB * NS

R = 1000
BN_EPS = 1e-5


def _sc_segment_sum(xrows, srcp, dstp):
    mesh = plsc.VectorSubcoreMesh(core_axis_name="c", subcore_axis_name="s")

    @functools.partial(
        pl.kernel,
        out_type=jax.ShapeDtypeStruct((NC * ACC_ROWS, CH), jnp.float32),
        mesh=mesh,
        scratch_types=[
            pltpu.VMEM((E_SUB,), jnp.int32),
            pltpu.VMEM((2, IDX, CHUNK), jnp.int32),
            pltpu.VMEM((CHUNK, CH), jnp.float32),
            pltpu.VMEM((CHUNK, CH), jnp.float32),
            pltpu.VMEM_SHARED((ACC_ROWS, CH), jnp.float32),
            pltpu.SemaphoreType.DMA,
            pltpu.SemaphoreType.DMA,
            pltpu.SemaphoreType.DMA,
        ],
    )
    def k(x_hbm, src_hbm, dst_hbm, out_hbm,
          src_v, dst_v, rows_a, rows_b, acc, sem_a, sem_b, sem_i):
        c = lax.axis_index("c")
        s = lax.axis_index("s")
        dbase = s * SUB_CHUNKS

        pltpu.async_copy(src_hbm.at[pl.ds(s * E_SUB, E_SUB)], src_v, sem_i)
        pltpu.async_copy(dst_hbm.at[pl.ds(dbase, IDX)], dst_v.at[0], sem_i)

        zero = jnp.zeros((L,), jnp.float32)

        @pl.loop(0, CHUNK)
        def _(r):
            @pl.loop(0, CH // L)
            def _(l):
                rows_a[r, pl.ds(l * L, L)] = zero

        for b in range(ROWS_SUB // CHUNK):
            pltpu.async_copy(
                rows_a, acc.at[pl.ds(s * ROWS_SUB + b * CHUNK, CHUNK)], sem_b)
        for b in range(ROWS_SUB // CHUNK):
            pltpu.make_async_copy(
                rows_a, acc.at[pl.ds(s * ROWS_SUB + b * CHUNK, CHUNK)],
                sem_b).wait()
        pltpu.make_async_copy(
            src_hbm.at[pl.ds(s * E_SUB, E_SUB)], src_v, sem_i).wait()
        pltpu.make_async_copy(
            dst_hbm.at[pl.ds(dbase, IDX)], dst_v.at[0], sem_i).wait()

        plsc.subcore_barrier()
        xv = x_hbm.at[:, pl.ds(c * CH, CH)]

        for t in range(N_STAGES):
            slot = t % 2
            dv = dst_v.at[slot]
            sb = t * IDX * CHUNK

            def sv(j):
                return src_v.at[pl.ds(sb + j * CHUNK, CHUNK)]

            if t > 0:
                pltpu.make_async_copy(
                    dst_hbm.at[pl.ds(dbase + t * IDX, IDX)],
                    dst_v.at[slot], sem_i).wait()
            if t + 1 < N_STAGES:
                pltpu.async_copy(
                    dst_hbm.at[pl.ds(dbase + (t + 1) * IDX, IDX)],
                    dst_v.at[1 - slot], sem_i)

            pltpu.async_copy(xv.at[sv(0)], rows_a, sem_a)

            @pl.loop(0, IDX, step=2)
            def _(j):
                pltpu.async_copy(xv.at[sv(j + 1)], rows_b, sem_b)
                pltpu.make_async_copy(
                    xv.at[sv(j)], rows_a, sem_a).wait()
                pltpu.sync_copy(rows_a, acc.at[dv.at[j]], add=True)

                @pl.when(j + 2 < IDX)
                def _():
                    pltpu.async_copy(xv.at[sv(j + 2)], rows_a, sem_a)

                pltpu.make_async_copy(
                    xv.at[sv(j + 1)], rows_b, sem_b).wait()
                pltpu.sync_copy(rows_b, acc.at[dv.at[j + 1]], add=True)

        plsc.subcore_barrier()

        pltpu.sync_copy(
            acc.at[pl.ds(s * ROWS_SUB, ROWS_SUB)],
            out_hbm.at[pl.ds(c * ACC_ROWS + s * ROWS_SUB, ROWS_SUB)])

    return k(xrows, srcp, dstp)


def _mlp(scale, x, agg3, W1, b1, g1, be1, W2, b2, g2, be2):
    def body(sc_ref, x_ref, a_ref, w1_ref, b1_ref, g1_ref, be1_ref,
             w2_ref, b2_ref, g2_ref, be2_ref, o_ref,
             h_s, y2_s, G_s, hs_s, m1_s, i1_s, s2_s, q2_s, m2_s, i2_s):
        p = pl.program_id(0)
        i = pl.program_id(1)

        @pl.when(p == 0)
        def _():
            sc = sc_ref[0, 0]
            h = jnp.concatenate(
                [sc * x_ref[:, :CH] + a_ref[0],
                 sc * x_ref[:, CH:] + a_ref[1]], axis=1)
            h_s[pl.ds(i * R, R), :] = h
            G = lax.dot_general(h, h, (((0,), (0,)), ((), ())),
                                preferred_element_type=jnp.float32)
            cs = jnp.sum(h, axis=0, keepdims=True)

            @pl.when(i == 0)
            def _():
                G_s[...] = G
                hs_s[...] = cs

            @pl.when(i != 0)
            def _():
                G_s[...] += G
                hs_s[...] += cs

        @pl.when(p == 1)
        def _():
            @pl.when(i == 0)
            def _():
                w1 = w1_ref[...]
                b1v = b1_ref[...]
                sw = lax.dot_general(hs_s[...], w1, (((1,), (0,)), ((), ())),
                                     preferred_element_type=jnp.float32)
                gw = lax.dot_general(G_s[...], w1, (((1,), (0,)), ((), ())),
                                     preferred_element_type=jnp.float32)
                q1 = (jnp.sum(w1 * gw, axis=0, keepdims=True)
                      + 2.0 * b1v * sw + N * b1v * b1v)
                s1 = sw + N * b1v
                m = s1 * (1.0 / N)
                v = q1 * (1.0 / N) - m * m
                m1_s[...] = m
                i1_s[...] = lax.rsqrt(v + BN_EPS) * g1_ref[...]

            h = h_s[pl.ds(i * R, R), :]
            y1 = lax.dot_general(h, w1_ref[...], (((1,), (0,)), ((), ())),
                                 preferred_element_type=jnp.float32)
            y1 = y1 + b1_ref[...]
            h1 = jnp.maximum((y1 - m1_s[...]) * i1_s[...] + be1_ref[...], 0.0)
            y2 = lax.dot_general(h1, w2_ref[...], (((1,), (0,)), ((), ())),
                                 preferred_element_type=jnp.float32)
            y2 = y2 + b2_ref[...]
            y2_s[pl.ds(i * R, R), :] = y2
            cs = jnp.sum(y2, axis=0, keepdims=True)
            cq = jnp.sum(y2 * y2, axis=0, keepdims=True)

            @pl.when(i == 0)
            def _():
                s2_s[...] = cs
                q2_s[...] = cq

            @pl.when(i != 0)
            def _():
                s2_s[...] += cs
                q2_s[...] += cq

        @pl.when(p == 2)
        def _():
            @pl.when(i == 0)
            def _():
                m = s2_s[...] * (1.0 / N)
                v = q2_s[...] * (1.0 / N) - m * m
                m2_s[...] = m
                i2_s[...] = lax.rsqrt(v + BN_EPS) * g2_ref[...]

            y2 = y2_s[pl.ds(i * R, R), :]
            o_ref[...] = jnp.maximum(
                (y2 - m2_s[...]) * i2_s[...] + be2_ref[...], 0.0)

    zero2 = lambda p, i: (0, 0)
    return pl.pallas_call(
        body,
        grid=(3, N // R),
        in_specs=[
            pl.BlockSpec((1, 1), zero2),
            pl.BlockSpec((R, C), lambda p, i: (jnp.where(p == 0, i, 0), 0)),
            pl.BlockSpec((2, R, CH),
                         lambda p, i: (0, jnp.where(p == 0, i, 0), 0)),
            pl.BlockSpec((C, H), zero2),
            pl.BlockSpec((1, H), zero2),
            pl.BlockSpec((1, H), zero2),
            pl.BlockSpec((1, H), zero2),
            pl.BlockSpec((H, C), zero2),
            pl.BlockSpec((1, C), zero2),
            pl.BlockSpec((1, C), zero2),
            pl.BlockSpec((1, C), zero2),
        ],
        out_specs=pl.BlockSpec((R, C),
                               lambda p, i: (jnp.where(p == 2, i, 0), 0)),
        out_shape=jax.ShapeDtypeStruct((N, C), jnp.float32),
        scratch_shapes=[
            pltpu.VMEM((N, C), jnp.float32),
            pltpu.VMEM((N, C), jnp.float32),
            pltpu.VMEM((C, C), jnp.float32),
            pltpu.VMEM((1, C), jnp.float32),
            pltpu.VMEM((1, H), jnp.float32),
            pltpu.VMEM((1, H), jnp.float32),
            pltpu.VMEM((1, C), jnp.float32),
            pltpu.VMEM((1, C), jnp.float32),
            pltpu.VMEM((1, C), jnp.float32),
            pltpu.VMEM((1, C), jnp.float32),
        ],
    )(scale, x, agg3, W1, b1, g1, be1, W2, b2, g2, be2)


def kernel(x, edge_index, eps, W1, b1, g1, be1, W2, b2, g2, be2):
    src = edge_index[0]
    dst = edge_index[1]

    pad_n = E_PAD - E
    spread = jnp.arange(pad_n, dtype=jnp.int32) % 8
    src_p = jnp.concatenate([src, spread])
    dst_p = jnp.concatenate(
        [dst, N + spread]).reshape(E_PAD // CHUNK, CHUNK)

    agg = _sc_segment_sum(x, src_p, dst_p)
    agg3 = agg.reshape(NC, ACC_ROWS, CH)

    scale = (1.0 + eps).reshape(1, 1).astype(jnp.float32)
    return _mlp(scale, x, agg3, W1, b1.reshape(1, H), g1.reshape(1, H),
                be1.reshape(1, H), W2, b2.reshape(1, C), g2.reshape(1, C),
                be2.reshape(1, C))
